# baseline (speedup 1.0000x reference)
import functools

import jax
import jax.numpy as jnp
from jax.experimental import pallas as pl


_N_TILES = 8


def _pool_kernel(x_ref, w_ref, b_ref, o_ref):
    x = x_ref[:, 0, :]
    w = w_ref[...]
    acc = jax.lax.dot_general(
        x, w,
        dimension_numbers=(((1,), (1,)), ((), ())),
        preferred_element_type=jnp.float32,
    )
    o_ref[...] = jnp.tanh(acc + b_ref[...])


@jax.jit
def kernel(hidden_states, W, b):
    B, _, H = hidden_states.shape
    O = W.shape[0]
    tile = O // _N_TILES

    b2 = b.reshape(1, O)

    out = pl.pallas_call(
        _pool_kernel,
        grid=(_N_TILES,),
        in_specs=[
            pl.BlockSpec((B, 8, H), lambda i: (0, 0, 0)),
            pl.BlockSpec((tile, H), lambda i: (i, 0)),
            pl.BlockSpec((1, tile), lambda i: (0, i)),
        ],
        out_specs=pl.BlockSpec((B, tile), lambda i: (0, i)),
        out_shape=jax.ShapeDtypeStruct((B, O), jnp.float32),
    )(hidden_states, W, b2)
    return out


# speedup vs baseline: 1.8414x; 1.8414x over previous
import functools

import jax
import jax.numpy as jnp
from jax.experimental import pallas as pl


_N_TILES = 1


def _pool_kernel(x_ref, w_ref, b_ref, o_ref):
    x = x_ref[:, 0, :]
    w = w_ref[...]
    acc = jax.lax.dot_general(
        x, w,
        dimension_numbers=(((1,), (1,)), ((), ())),
        preferred_element_type=jnp.float32,
    )
    o_ref[...] = jnp.tanh(acc + b_ref[...])


@jax.jit
def kernel(hidden_states, W, b):
    B, _, H = hidden_states.shape
    O = W.shape[0]
    tile = O // _N_TILES

    b2 = b.reshape(1, O)

    out = pl.pallas_call(
        _pool_kernel,
        grid=(_N_TILES,),
        in_specs=[
            pl.BlockSpec((B, 8, H), lambda i: (0, 0, 0)),
            pl.BlockSpec((tile, H), lambda i: (i, 0)),
            pl.BlockSpec((1, tile), lambda i: (0, i)),
        ],
        out_specs=pl.BlockSpec((B, tile), lambda i: (0, i)),
        out_shape=jax.ShapeDtypeStruct((B, O), jnp.float32),
    )(hidden_states, W, b2)
    return out
